# Initial kernel scaffold; baseline (speedup 1.0000x reference)
import jax
import jax.numpy as jnp
from jax.experimental import pallas as pl


def kernel(x, table):
    raise NotImplementedError("write your pallas kernel here")


# speedup vs baseline: 6.3648x; 6.3648x over previous
import functools

import jax
import jax.numpy as jnp
from jax import lax
from jax.experimental import pallas as pl
from jax.experimental.pallas import tpu as pltpu
from jax.experimental.pallas import tpu_sc as plsc

HASH_BUCKETS = 250000
EMBED_DIM = 64
CHUNK = 128
LANES = 16


def _emb_body(n_chunks, x_hbm, table_hbm, out_hbm, idx_v, rows_v, sem):
    nc = lax.axis_index("c")
    ns = lax.axis_index("s")
    wid = ns * 2 + nc
    base = wid * (n_chunks * CHUNK)

    def body(i, carry):
        row0 = base + i * CHUNK
        pltpu.sync_copy(x_hbm.at[pl.ds(row0, CHUNK)], idx_v)
        for j in range(CHUNK // LANES):
            sl = pl.ds(j * LANES, LANES)
            idx_v[sl] = lax.rem(idx_v[sl], HASH_BUCKETS)
        pltpu.async_copy(table_hbm.at[idx_v], rows_v, sem).wait()
        pltpu.sync_copy(rows_v, out_hbm.at[pl.ds(row0, CHUNK)])
        return carry

    lax.fori_loop(0, n_chunks, body, 0)


def kernel(x, table):
    rows, cols = x.shape
    b = rows * cols
    xf = x.reshape(b).astype(jnp.int32)
    nw = 32
    assert b % (nw * CHUNK) == 0
    n_chunks = b // (nw * CHUNK)

    mesh = plsc.VectorSubcoreMesh(core_axis_name="c", subcore_axis_name="s")
    run = functools.partial(
        pl.kernel,
        mesh=mesh,
        compiler_params=pltpu.CompilerParams(use_tc_tiling_on_sc=False),
        out_type=jax.ShapeDtypeStruct((b, EMBED_DIM), jnp.float32),
        scratch_types=[
            pltpu.VMEM((CHUNK,), jnp.int32),
            pltpu.VMEM((CHUNK, EMBED_DIM), jnp.float32),
            pltpu.SemaphoreType.DMA,
        ],
    )(functools.partial(_emb_body, n_chunks))
    out = run(xf, table)
    return out.reshape(rows, cols, EMBED_DIM)


# speedup vs baseline: 9.1550x; 1.4384x over previous
import functools

import jax
import jax.numpy as jnp
from jax import lax
from jax.experimental import pallas as pl
from jax.experimental.pallas import tpu as pltpu
from jax.experimental.pallas import tpu_sc as plsc

HASH_BUCKETS = 250000
EMBED_DIM = 64
CHUNK = 128
K = 4
GROUP = K * CHUNK
LANES = 16
NW = 32


def _emb_body(n_groups, x_hbm, table_hbm, out_hbm,
              idx_a, idx_b, rows_a, rows_b,
              gsem_a, gsem_b, wsem_a, wsem_b):
    wid = lax.axis_index("s") * 2 + lax.axis_index("c")
    base_row = wid * (n_groups * GROUP)

    def load_idx(idx_s, g):
        row0 = base_row + g * GROUP
        pltpu.sync_copy(x_hbm.at[pl.ds(row0, GROUP)], idx_s)

        def mod_one(j, carry):
            sl = pl.ds(j * LANES, LANES)
            idx_s[sl] = lax.rem(idx_s[sl], HASH_BUCKETS)
            return carry

        lax.fori_loop(0, GROUP // LANES, mod_one, 0)

    def fire_gathers(idx_s, rows_s, gsem):
        for b in range(K):
            pltpu.async_copy(table_hbm.at[idx_s.at[pl.ds(b * CHUNK, CHUNK)]],
                             rows_s.at[pl.ds(b * CHUNK, CHUNK)], gsem)

    def drain_gathers(rows_s, gsem):
        pltpu.make_async_copy(out_hbm.at[pl.ds(base_row, GROUP)], rows_s,
                              gsem).wait()

    def fire_write(rows_s, wsem, g):
        row0 = base_row + g * GROUP
        pltpu.async_copy(rows_s, out_hbm.at[pl.ds(row0, GROUP)], wsem)

    def wait_write(rows_s, wsem):
        pltpu.make_async_copy(out_hbm.at[pl.ds(base_row, GROUP)], rows_s,
                              wsem).wait()

    load_idx(idx_a, 0)
    fire_gathers(idx_a, rows_a, gsem_a)
    load_idx(idx_b, 1)
    fire_gathers(idx_b, rows_b, gsem_b)

    def pair(i, carry):
        drain_gathers(rows_a, gsem_a)
        fire_write(rows_a, wsem_a, 2 * i)
        load_idx(idx_a, 2 * i + 2)
        wait_write(rows_a, wsem_a)
        fire_gathers(idx_a, rows_a, gsem_a)
        drain_gathers(rows_b, gsem_b)
        fire_write(rows_b, wsem_b, 2 * i + 1)
        load_idx(idx_b, 2 * i + 3)
        wait_write(rows_b, wsem_b)
        fire_gathers(idx_b, rows_b, gsem_b)
        return carry

    n_pairs = n_groups // 2
    lax.fori_loop(0, n_pairs - 1, pair, 0)

    drain_gathers(rows_a, gsem_a)
    fire_write(rows_a, wsem_a, n_groups - 2)
    drain_gathers(rows_b, gsem_b)
    fire_write(rows_b, wsem_b, n_groups - 1)
    wait_write(rows_a, wsem_a)
    wait_write(rows_b, wsem_b)


def kernel(x, table):
    rows, cols = x.shape
    b = rows * cols
    xf = x.reshape(b).astype(jnp.int32)
    assert b % (NW * GROUP) == 0
    n_groups = b // (NW * GROUP)
    assert n_groups % 2 == 0

    mesh = plsc.VectorSubcoreMesh(core_axis_name="c", subcore_axis_name="s")
    run = functools.partial(
        pl.kernel,
        mesh=mesh,
        compiler_params=pltpu.CompilerParams(use_tc_tiling_on_sc=False),
        out_type=jax.ShapeDtypeStruct((b, EMBED_DIM), jnp.float32),
        scratch_types=(
            [pltpu.VMEM((GROUP,), jnp.int32) for _ in range(2)]
            + [pltpu.VMEM((GROUP, EMBED_DIM), jnp.float32) for _ in range(2)]
            + [pltpu.SemaphoreType.DMA for _ in range(4)]
        ),
    )(functools.partial(_emb_body, n_groups))
    out = run(xf, table)
    return out.reshape(rows, cols, EMBED_DIM)
